# baseline (speedup 1.0000x reference)
import jax
import jax.numpy as jnp
from jax.experimental import pallas as pl
from jax.experimental.pallas import tpu as pltpu

_NUM_SAMPLES = 100000
_NUM_FEATURES = 32
_BATCH = 1024
_INV_TEMP = 20.0
_BLOCK_N = 2048


def _mm_kernel(x_ref, f_ref, o_ref):
    x = x_ref[...]
    nrm = jnp.sqrt(jnp.sum(x * x, axis=1, keepdims=True))
    x = x * (_INV_TEMP / jnp.clip(nrm, 1e-12, None))
    o_ref[...] = jax.lax.dot_general(
        x, f_ref[...], (((1,), (1,)), ((), ())),
        preferred_element_type=jnp.float32)


def kernel(inputs, targets, features):
    del targets
    grid = (pl.cdiv(_NUM_SAMPLES, _BLOCK_N),)
    return pl.pallas_call(
        _mm_kernel,
        grid=grid,
        in_specs=[
            pl.BlockSpec((_BATCH, _NUM_FEATURES), lambda i: (0, 0)),
            pl.BlockSpec((_BLOCK_N, _NUM_FEATURES), lambda i: (i, 0)),
        ],
        out_specs=pl.BlockSpec((_BATCH, _BLOCK_N), lambda i: (0, i)),
        out_shape=jax.ShapeDtypeStruct((_BATCH, _NUM_SAMPLES), jnp.float32),
        compiler_params=pltpu.CompilerParams(
            dimension_semantics=("arbitrary",)),
    )(inputs, features)
